# Initial kernel scaffold; baseline (speedup 1.0000x reference)
import jax
import jax.numpy as jnp
from jax.experimental import pallas as pl


def kernel(x_nchw, w1, b1, w2, b2, w3, b3, w4, b4, w5, b5, wout, bout):
    raise NotImplementedError("write your pallas kernel here")


# speedup vs baseline: 37.4887x; 37.4887x over previous
import jax
import jax.numpy as jnp
from jax.experimental import pallas as pl
from jax.experimental.pallas import tpu as pltpu

_VMEM = 32 * 1024 * 1024


def _block_rows(w, cin):
    cols = w.shape[1]
    v = w.reshape(2, 2, 2, 2, cin, cols)
    return v.transpose(0, 2, 1, 3, 4, 5).reshape(16 * cin, cols)


def _net_body(xb_ref, w1_ref, b1_ref, w2_ref, b2_ref, w3_ref, b3_ref,
              w4_ref, b4_ref, w5_ref, b5_ref, o_ref, s1_ref, s2_ref, s3_ref):
    def conv_pool(xb, w, b, cout):
        hb, wb, _ = xb.shape
        hp, wp = hb - 1, wb - 1
        rp = jnp.concatenate(
            [xb[di:di + hp, dj:dj + wp, :] for di in (0, 1) for dj in (0, 1)],
            axis=-1)
        k = rp.shape[-1]
        acc = jnp.dot(rp.reshape(hp * wp, k), w,
                      preferred_element_type=jnp.float32) + b
        m = jnp.maximum(
            jnp.maximum(acc[:, 0 * cout:1 * cout], acc[:, 1 * cout:2 * cout]),
            jnp.maximum(acc[:, 2 * cout:3 * cout], acc[:, 3 * cout:4 * cout]))
        return jnp.maximum(m, 0.0).reshape(hp, wp, cout)

    def to_blocks(a, s_ref):
        hp, wp, c = a.shape
        s_ref[...] = a.reshape(hp // 2, 2, wp, c)
        parts = [s_ref[:, dy, pl.ds(dx, wp // 2, stride=2), :]
                 for dy in (0, 1) for dx in (0, 1)]
        return jnp.concatenate(parts, axis=-1)

    x = xb_ref[0]
    a = conv_pool(x, w1_ref[...], b1_ref[...], 16)
    a = conv_pool(to_blocks(a, s1_ref), w2_ref[...], b2_ref[...], 32)
    a = conv_pool(to_blocks(a, s2_ref), w3_ref[...], b3_ref[...], 64)
    a = conv_pool(to_blocks(a, s3_ref), w4_ref[...], b4_ref[...], 128)
    y = jnp.dot(a.reshape(55, 128), w5_ref[...],
                preferred_element_type=jnp.float32) + b5_ref[...]
    o_ref[0] = jnp.maximum(y, 0.0)


def _convnet(xb, w1p, b1, w2p, b2, w3p, b3, w4p, b4, w5, b5):
    n = xb.shape[0]
    return pl.pallas_call(
        _net_body,
        out_shape=jax.ShapeDtypeStruct((n, 55, 64), jnp.float32),
        grid=(n,),
        in_specs=[
            pl.BlockSpec((1, 55, 103, 12), lambda i: (i, 0, 0, 0)),
            pl.BlockSpec((48, 64), lambda i: (0, 0)),
            pl.BlockSpec((1, 64), lambda i: (0, 0)),
            pl.BlockSpec((256, 128), lambda i: (0, 0)),
            pl.BlockSpec((1, 128), lambda i: (0, 0)),
            pl.BlockSpec((512, 256), lambda i: (0, 0)),
            pl.BlockSpec((1, 256), lambda i: (0, 0)),
            pl.BlockSpec((1024, 512), lambda i: (0, 0)),
            pl.BlockSpec((1, 512), lambda i: (0, 0)),
            pl.BlockSpec((128, 64), lambda i: (0, 0)),
            pl.BlockSpec((1, 64), lambda i: (0, 0)),
        ],
        out_specs=pl.BlockSpec((1, 55, 64), lambda i: (i, 0, 0)),
        scratch_shapes=[
            pltpu.VMEM((27, 2, 102, 16), jnp.float32),
            pltpu.VMEM((13, 2, 50, 32), jnp.float32),
            pltpu.VMEM((6, 2, 24, 64), jnp.float32),
        ],
        compiler_params=pltpu.CompilerParams(
            dimension_semantics=("parallel",),
            vmem_limit_bytes=_VMEM,
        ),
    )(xb, w1p, b1, w2p, b2, w3p, b3, w4p, b4, w5, b5)


def _linear_body(x_ref, w_ref, b_ref, o_ref):
    o_ref[...] = (
        jnp.dot(x_ref[...], w_ref[...], preferred_element_type=jnp.float32)
        + b_ref[...])


def _linear(xf, w, b):
    n, k = xf.shape
    cols = w.shape[1]
    bm = n // 2
    return pl.pallas_call(
        _linear_body,
        out_shape=jax.ShapeDtypeStruct((n, cols), jnp.float32),
        grid=(2,),
        in_specs=[pl.BlockSpec((bm, k), lambda i: (i, 0)),
                  pl.BlockSpec((k, cols), lambda i: (0, 0)),
                  pl.BlockSpec((1, cols), lambda i: (0, 0))],
        out_specs=pl.BlockSpec((bm, cols), lambda i: (i, 0)),
        compiler_params=pltpu.CompilerParams(
            dimension_semantics=("parallel",),
            vmem_limit_bytes=_VMEM,
        ),
    )(xf, w, b)


def kernel(x_nchw, w1, b1, w2, b2, w3, b3, w4, b4, w5, b5, wout, bout):
    n = x_nchw.shape[0]
    xt = jnp.transpose(x_nchw, (0, 2, 3, 1))
    xb = (xt.reshape(n, 55, 2, 103, 2, 3)
          .transpose(0, 1, 3, 2, 4, 5)
          .reshape(n, 55, 103, 12))
    w1p = _block_rows(w1, 3)
    w2p = _block_rows(w2, 16)
    w3p = _block_rows(w3, 32)
    w4p = _block_rows(w4, 64)
    feats = _convnet(xb, w1p, b1, w2p, b2, w3p, b3, w4p, b4, w5, b5)
    xf = feats.reshape(n, 55 * 64)
    out = _linear(xf, wout, bout)
    return out[:, :716]


# speedup vs baseline: 44.7128x; 1.1927x over previous
import jax
import jax.numpy as jnp
from jax.experimental import pallas as pl
from jax.experimental.pallas import tpu as pltpu

_VMEM = 32 * 1024 * 1024


def _block_rows(w, cin):
    cols = w.shape[1]
    v = w.reshape(2, 2, 2, 2, cin, cols)
    return v.transpose(0, 2, 1, 3, 4, 5).reshape(16 * cin, cols)


def _net_body(xb_ref, w1_ref, b1_ref, w2_ref, b2_ref, w3_ref, b3_ref,
              w4_ref, b4_ref, w5_ref, b5_ref, o_ref,
              s1, b2s, s2, b3s, s3, b4s):
    f32 = jnp.float32

    def pool_relu(acc, cout):
        m = jnp.maximum(
            jnp.maximum(acc[:, 0 * cout:1 * cout], acc[:, 1 * cout:2 * cout]),
            jnp.maximum(acc[:, 2 * cout:3 * cout], acc[:, 3 * cout:4 * cout]))
        return jnp.maximum(m, 0.0)

    def halve(s_ref, dy, dx, wq):
        return s_ref[:, dy, pl.ds(dx, wq, stride=2), :]

    rp = jnp.concatenate(
        [xb_ref[0, di:di + 54, dj:dj + 104, :] for di in (0, 1)
         for dj in (0, 1)], axis=-1)
    acc = jnp.dot(rp.reshape(54 * 104, 48), w1_ref[...],
                  preferred_element_type=f32) + b1_ref[...]
    s1[...] = pool_relu(acc, 16).reshape(27, 2, 104, 16)

    b2s[:, 52:64, :] = jnp.zeros((27, 12, 64), jnp.bfloat16)
    b2s[:, 0:52, :] = jnp.concatenate(
        [halve(s1, dy, dx, 52) for dy in (0, 1) for dx in (0, 1)],
        axis=-1).astype(jnp.bfloat16)
    rp = jnp.concatenate(
        [b2s[di:di + 26, dj:dj + 56, :] for di in (0, 1) for dj in (0, 1)],
        axis=-1)
    acc = jnp.dot(rp.reshape(26 * 56, 256), w2_ref[...],
                  preferred_element_type=f32) + b2_ref[...]
    s2[...] = pool_relu(acc, 32).reshape(13, 2, 56, 32)

    b3s[...] = jnp.concatenate(
        [halve(s2, dy, dx, 28) for dy in (0, 1) for dx in (0, 1)],
        axis=-1).astype(jnp.bfloat16)
    rp = jnp.concatenate(
        [b3s[di:di + 12, dj:dj + 24, :] for di in (0, 1) for dj in (0, 1)],
        axis=-1)
    acc = jnp.dot(rp.reshape(12 * 24, 512), w3_ref[...],
                  preferred_element_type=f32) + b3_ref[...]
    s3[...] = pool_relu(acc, 64).reshape(6, 2, 24, 64)

    b4s[...] = jnp.concatenate(
        [halve(s3, dy, dx, 12) for dy in (0, 1) for dx in (0, 1)],
        axis=-1).astype(jnp.bfloat16)
    rp = jnp.concatenate(
        [b4s[di:di + 5, dj:dj + 11, :] for di in (0, 1) for dj in (0, 1)],
        axis=-1)
    acc = jnp.dot(rp.reshape(55, 1024), w4_ref[...],
                  preferred_element_type=f32) + b4_ref[...]
    m = pool_relu(acc, 128).astype(jnp.bfloat16)
    y = jnp.dot(m, w5_ref[...], preferred_element_type=f32) + b5_ref[...]
    o_ref[0] = jnp.maximum(y, 0.0).astype(jnp.bfloat16)


def _convnet(xb, w1p, b1, w2p, b2, w3p, b3, w4p, b4, w5, b5):
    n = xb.shape[0]
    bf16 = jnp.bfloat16
    return pl.pallas_call(
        _net_body,
        out_shape=jax.ShapeDtypeStruct((n, 55, 64), bf16),
        grid=(n,),
        in_specs=[
            pl.BlockSpec((1, 55, 106, 12), lambda i: (i, 0, 0, 0)),
            pl.BlockSpec((48, 64), lambda i: (0, 0)),
            pl.BlockSpec((1, 64), lambda i: (0, 0)),
            pl.BlockSpec((256, 128), lambda i: (0, 0)),
            pl.BlockSpec((1, 128), lambda i: (0, 0)),
            pl.BlockSpec((512, 256), lambda i: (0, 0)),
            pl.BlockSpec((1, 256), lambda i: (0, 0)),
            pl.BlockSpec((1024, 512), lambda i: (0, 0)),
            pl.BlockSpec((1, 512), lambda i: (0, 0)),
            pl.BlockSpec((128, 64), lambda i: (0, 0)),
            pl.BlockSpec((1, 64), lambda i: (0, 0)),
        ],
        out_specs=pl.BlockSpec((1, 55, 64), lambda i: (i, 0, 0)),
        scratch_shapes=[
            pltpu.VMEM((27, 2, 104, 16), jnp.float32),
            pltpu.VMEM((27, 64, 64), bf16),
            pltpu.VMEM((13, 2, 56, 32), jnp.float32),
            pltpu.VMEM((13, 28, 128), bf16),
            pltpu.VMEM((6, 2, 24, 64), jnp.float32),
            pltpu.VMEM((6, 12, 256), bf16),
        ],
        compiler_params=pltpu.CompilerParams(
            dimension_semantics=("parallel",),
            vmem_limit_bytes=_VMEM,
        ),
    )(xb, w1p, b1, w2p, b2, w3p, b3, w4p, b4, w5, b5)


def _linear_body(x_ref, w_ref, b_ref, o_ref):
    o_ref[...] = (
        jnp.dot(x_ref[...], w_ref[...], preferred_element_type=jnp.float32)
        + b_ref[...])


def _linear(xf, w, b):
    n, k = xf.shape
    cols = w.shape[1]
    bm = n // 2
    return pl.pallas_call(
        _linear_body,
        out_shape=jax.ShapeDtypeStruct((n, cols), jnp.float32),
        grid=(2,),
        in_specs=[pl.BlockSpec((bm, k), lambda i: (i, 0)),
                  pl.BlockSpec((k, cols), lambda i: (0, 0)),
                  pl.BlockSpec((1, cols), lambda i: (0, 0))],
        out_specs=pl.BlockSpec((bm, cols), lambda i: (i, 0)),
        compiler_params=pltpu.CompilerParams(
            dimension_semantics=("parallel",),
            vmem_limit_bytes=_VMEM,
        ),
    )(xf, w, b)


def kernel(x_nchw, w1, b1, w2, b2, w3, b3, w4, b4, w5, b5, wout, bout):
    n = x_nchw.shape[0]
    bf16 = jnp.bfloat16
    xt = jnp.transpose(x_nchw, (0, 2, 3, 1))
    xb = (xt.reshape(n, 55, 2, 103, 2, 3)
          .transpose(0, 1, 3, 2, 4, 5)
          .reshape(n, 55, 103, 12))
    xb = jnp.pad(xb, ((0, 0), (0, 0), (0, 3), (0, 0))).astype(bf16)
    w1p = _block_rows(w1, 3).astype(bf16)
    w2p = _block_rows(w2, 16).astype(bf16)
    w3p = _block_rows(w3, 32).astype(bf16)
    w4p = _block_rows(w4, 64).astype(bf16)
    feats = _convnet(xb, w1p, b1, w2p, b2, w3p, b3, w4p, b4,
                     w5.astype(bf16), b5)
    xf = feats.reshape(n, 55 * 64)
    out = _linear(xf, wout.astype(bf16), bout)
    return out[:, :716]


# speedup vs baseline: 51.8870x; 1.1605x over previous
import jax
import jax.numpy as jnp
from jax.experimental import pallas as pl
from jax.experimental.pallas import tpu as pltpu

_VMEM = 48 * 1024 * 1024
_IMG = 2


def _block_rows(w, cin):
    cols = w.shape[1]
    v = w.reshape(2, 2, 2, 2, cin, cols)
    return v.transpose(0, 2, 1, 3, 4, 5).reshape(16 * cin, cols)


def _net_body(xb_ref, w1_ref, b1_ref, w2_ref, b2_ref, w3_ref, b3_ref,
              w4_ref, b4_ref, w5_ref, b5_ref, o_ref,
              s1, b2s, s2, b3s, s3, b4s):
    f32 = jnp.float32
    bf16 = jnp.bfloat16
    g = _IMG

    def pool_relu(acc, cout):
        m = jnp.maximum(
            jnp.maximum(acc[:, 0 * cout:1 * cout], acc[:, 1 * cout:2 * cout]),
            jnp.maximum(acc[:, 2 * cout:3 * cout], acc[:, 3 * cout:4 * cout]))
        return jnp.maximum(m, 0.0)

    def halve(s_ref, dy, dx, wq):
        return s_ref[:, dy, pl.ds(dx, wq, stride=2), :]

    rp = jnp.concatenate(
        [xb_ref[:, di:di + 54, dj:dj + 104, :] for di in (0, 1)
         for dj in (0, 1)], axis=-1)
    acc = jnp.dot(rp.reshape(g * 54 * 104, 48), w1_ref[...],
                  preferred_element_type=f32) + b1_ref[...]
    s1[...] = pool_relu(acc, 16).reshape(g * 27, 2, 104, 16)

    b2s[:, :, 52:64, :] = jnp.zeros((g, 27, 12, 64), bf16)
    b2s[:, :, 0:52, :] = jnp.concatenate(
        [halve(s1, dy, dx, 52) for dy in (0, 1) for dx in (0, 1)],
        axis=-1).astype(bf16).reshape(g, 27, 52, 64)
    rp = jnp.concatenate(
        [b2s[:, di:di + 26, dj:dj + 56, :] for di in (0, 1) for dj in (0, 1)],
        axis=-1)
    acc = jnp.dot(rp.reshape(g * 26 * 56, 256), w2_ref[...],
                  preferred_element_type=f32) + b2_ref[...]
    s2[...] = pool_relu(acc, 32).reshape(g * 13, 2, 56, 32)

    b3s[...] = jnp.concatenate(
        [halve(s2, dy, dx, 28) for dy in (0, 1) for dx in (0, 1)],
        axis=-1).astype(bf16).reshape(g, 13, 28, 128)
    rp = jnp.concatenate(
        [b3s[:, di:di + 12, dj:dj + 24, :] for di in (0, 1) for dj in (0, 1)],
        axis=-1)
    acc = jnp.dot(rp.reshape(g * 12 * 24, 512), w3_ref[...],
                  preferred_element_type=f32) + b3_ref[...]
    s3[...] = pool_relu(acc, 64).reshape(g * 6, 2, 24, 64)

    b4s[...] = jnp.concatenate(
        [halve(s3, dy, dx, 12) for dy in (0, 1) for dx in (0, 1)],
        axis=-1).astype(bf16).reshape(g, 6, 12, 256)
    rp = jnp.concatenate(
        [b4s[:, di:di + 5, dj:dj + 11, :] for di in (0, 1) for dj in (0, 1)],
        axis=-1)
    acc = jnp.dot(rp.reshape(g * 55, 1024), w4_ref[...],
                  preferred_element_type=f32) + b4_ref[...]
    m = pool_relu(acc, 128).astype(bf16)
    y = jnp.dot(m, w5_ref[...], preferred_element_type=f32) + b5_ref[...]
    o_ref[...] = jnp.maximum(y, 0.0).astype(bf16).reshape(g, 55, 64)


def _convnet(xb, w1p, b1, w2p, b2, w3p, b3, w4p, b4, w5, b5):
    n = xb.shape[0]
    g = _IMG
    bf16 = jnp.bfloat16
    return pl.pallas_call(
        _net_body,
        out_shape=jax.ShapeDtypeStruct((n, 55, 64), bf16),
        grid=(n // g,),
        in_specs=[
            pl.BlockSpec((g, 55, 106, 12), lambda i: (i, 0, 0, 0)),
            pl.BlockSpec((48, 64), lambda i: (0, 0)),
            pl.BlockSpec((1, 64), lambda i: (0, 0)),
            pl.BlockSpec((256, 128), lambda i: (0, 0)),
            pl.BlockSpec((1, 128), lambda i: (0, 0)),
            pl.BlockSpec((512, 256), lambda i: (0, 0)),
            pl.BlockSpec((1, 256), lambda i: (0, 0)),
            pl.BlockSpec((1024, 512), lambda i: (0, 0)),
            pl.BlockSpec((1, 512), lambda i: (0, 0)),
            pl.BlockSpec((128, 64), lambda i: (0, 0)),
            pl.BlockSpec((1, 64), lambda i: (0, 0)),
        ],
        out_specs=pl.BlockSpec((g, 55, 64), lambda i: (i, 0, 0)),
        scratch_shapes=[
            pltpu.VMEM((g * 27, 2, 104, 16), jnp.float32),
            pltpu.VMEM((g, 27, 64, 64), bf16),
            pltpu.VMEM((g * 13, 2, 56, 32), jnp.float32),
            pltpu.VMEM((g, 13, 28, 128), bf16),
            pltpu.VMEM((g * 6, 2, 24, 64), jnp.float32),
            pltpu.VMEM((g, 6, 12, 256), bf16),
        ],
        compiler_params=pltpu.CompilerParams(
            dimension_semantics=("parallel",),
            vmem_limit_bytes=_VMEM,
        ),
    )(xb, w1p, b1, w2p, b2, w3p, b3, w4p, b4, w5, b5)


def _linear_body(x_ref, w_ref, b_ref, o_ref):
    o_ref[...] = (
        jnp.dot(x_ref[...], w_ref[...], preferred_element_type=jnp.float32)
        + b_ref[...])


def _linear(xf, w, b):
    n, k = xf.shape
    cols = w.shape[1]
    bm = n // 2
    return pl.pallas_call(
        _linear_body,
        out_shape=jax.ShapeDtypeStruct((n, cols), jnp.float32),
        grid=(2,),
        in_specs=[pl.BlockSpec((bm, k), lambda i: (i, 0)),
                  pl.BlockSpec((k, cols), lambda i: (0, 0)),
                  pl.BlockSpec((1, cols), lambda i: (0, 0))],
        out_specs=pl.BlockSpec((bm, cols), lambda i: (i, 0)),
        compiler_params=pltpu.CompilerParams(
            dimension_semantics=("parallel",),
            vmem_limit_bytes=_VMEM,
        ),
    )(xf, w, b)


def kernel(x_nchw, w1, b1, w2, b2, w3, b3, w4, b4, w5, b5, wout, bout):
    n = x_nchw.shape[0]
    bf16 = jnp.bfloat16
    xb = (x_nchw.reshape(n, 3, 55, 2, 103, 2)
          .transpose(0, 2, 4, 3, 5, 1)
          .reshape(n, 55, 103, 12))
    xb = jnp.pad(xb, ((0, 0), (0, 0), (0, 3), (0, 0))).astype(bf16)
    w1p = _block_rows(w1, 3).astype(bf16)
    w2p = _block_rows(w2, 16).astype(bf16)
    w3p = _block_rows(w3, 32).astype(bf16)
    w4p = _block_rows(w4, 64).astype(bf16)
    feats = _convnet(xb, w1p, b1, w2p, b2, w3p, b3, w4p, b4,
                     w5.astype(bf16), b5)
    xf = feats.reshape(n, 55 * 64)
    out = _linear(xf, wout.astype(bf16), bout)
    return out[:, :716]
